# baseline (speedup 1.0000x reference)
import functools

import jax
import jax.numpy as jnp
from jax.experimental import pallas as pl
from jax.experimental.pallas import tpu as pltpu

TOPK = 5
_BIG = 2**30


def _retrieve_kernel(q_ref, k_ref, n_ref, out_v_ref, out_i_ref,
                     s_scr, cand_v, cand_i, *, nblk, blk, topk):
    i = pl.program_id(0)

    @pl.when(i == 0)
    def _init():
        out_v_ref[...] = jnp.full(out_v_ref.shape, -jnp.inf, jnp.float32)
        out_i_ref[...] = jnp.zeros(out_i_ref.shape, jnp.int32)

    kn = k_ref[...] / (n_ref[...] + 1e-8)
    s = jax.lax.dot_general(
        q_ref[...], kn, (((1,), (1,)), ((), ())),
        preferred_element_type=jnp.float32,
    )

    gbase = i * blk
    iota_b = jax.lax.broadcasted_iota(jnp.int32, s.shape, 1)

    thresh = out_v_ref[:, topk - 1][:, None]
    rowcnt = jnp.sum((s > thresh).astype(jnp.int32), axis=1, keepdims=True)
    mc = jnp.max(rowcnt)

    @pl.when(mc > 0)
    def _update():
        s_scr[...] = s
        cand_v[...] = jnp.full(cand_v.shape, -jnp.inf, jnp.float32)
        cand_i[...] = jnp.zeros(cand_i.shape, jnp.int32)

        for j in range(topk):
            @pl.when(j < mc)
            def _extract(j=j):
                sv = s_scr[...]
                m = jnp.max(sv, axis=1)
                idxc = jnp.where(sv == m[:, None], iota_b, _BIG)
                a = jnp.min(idxc, axis=1)
                cand_v[:, j:j + 1] = m[:, None]
                cand_i[:, j:j + 1] = a[:, None] + gbase
                s_scr[...] = jnp.where(idxc == a[:, None], -jnp.inf, sv)

        cv = jnp.concatenate([out_v_ref[...], cand_v[...]], axis=1)
        ci = jnp.concatenate([out_i_ref[...], cand_i[...]], axis=1)
        iota_c = jax.lax.broadcasted_iota(jnp.int32, cv.shape, 1)
        nv_cols, ni_cols = [], []
        for _ in range(topk):
            m = jnp.max(cv, axis=1)
            idxc = jnp.where(cv == m[:, None], iota_c, _BIG)
            a = jnp.min(idxc, axis=1)
            oh = idxc == a[:, None]
            sel = jnp.sum(jnp.where(oh, ci, 0), axis=1)
            nv_cols.append(m[:, None])
            ni_cols.append(sel[:, None])
            cv = jnp.where(oh, -jnp.inf, cv)
        out_v_ref[...] = jnp.concatenate(nv_cols, axis=1)
        out_i_ref[...] = jnp.concatenate(ni_cols, axis=1)


def _retrieve(qn, keys, knorm, blk, interpret=False):
    Q, D = qn.shape
    K, _ = keys.shape
    assert K % blk == 0
    nblk = K // blk
    kfn = functools.partial(_retrieve_kernel, nblk=nblk, blk=blk, topk=TOPK)
    return pl.pallas_call(
        kfn,
        grid=(nblk,),
        in_specs=[
            pl.BlockSpec((Q, D), lambda i: (0, 0)),
            pl.BlockSpec((blk, D), lambda i: (i, 0)),
            pl.BlockSpec((blk, 1), lambda i: (i, 0)),
        ],
        out_specs=[
            pl.BlockSpec((Q, TOPK), lambda i: (0, 0)),
            pl.BlockSpec((Q, TOPK), lambda i: (0, 0)),
        ],
        out_shape=[
            jax.ShapeDtypeStruct((Q, TOPK), jnp.float32),
            jax.ShapeDtypeStruct((Q, TOPK), jnp.int32),
        ],
        scratch_shapes=[
            pltpu.VMEM((Q, blk), jnp.float32),
            pltpu.VMEM((Q, TOPK), jnp.float32),
            pltpu.VMEM((Q, TOPK), jnp.int32),
        ],
        interpret=interpret,
    )(qn, keys, knorm)


@jax.jit
def kernel(queries, keys):
    qn = queries / (jnp.linalg.norm(queries, axis=-1, keepdims=True) + 1e-8)
    knorm = jnp.linalg.norm(keys, axis=-1, keepdims=True)
    K = keys.shape[0]
    blk = 2000 if K % 2000 == 0 else K
    return _retrieve(qn, keys, knorm, blk)


# speedup vs baseline: 1.4115x; 1.4115x over previous
import functools

import jax
import jax.numpy as jnp
from jax.experimental import pallas as pl
from jax.experimental.pallas import tpu as pltpu

TOPK = 5
_BIG = 2**30


def _retrieve_kernel(q_ref, k_ref, out_v_ref, out_i_ref,
                     s_scr, cand_v, cand_i, *, nblk, blk, topk):
    i = pl.program_id(0)

    @pl.when(i == 0)
    def _init():
        out_v_ref[...] = jnp.full(out_v_ref.shape, -jnp.inf, jnp.float32)
        out_i_ref[...] = jnp.zeros(out_i_ref.shape, jnp.int32)

    kb = k_ref[...]
    norm = jnp.sqrt(jnp.sum(kb * kb, axis=1, keepdims=True))
    kn = kb / (norm + 1e-8)
    s = jax.lax.dot_general(
        q_ref[...], kn, (((1,), (1,)), ((), ())),
        preferred_element_type=jnp.float32,
    )

    gbase = i * blk
    iota_b = jax.lax.broadcasted_iota(jnp.int32, s.shape, 1)

    thresh = out_v_ref[:, topk - 1][:, None]
    rowcnt = jnp.sum((s > thresh).astype(jnp.int32), axis=1, keepdims=True)
    mc = jnp.max(rowcnt)

    @pl.when(mc > 0)
    def _update():
        s_scr[...] = s
        cand_v[...] = jnp.full(cand_v.shape, -jnp.inf, jnp.float32)
        cand_i[...] = jnp.zeros(cand_i.shape, jnp.int32)

        for j in range(topk):
            @pl.when(j < mc)
            def _extract(j=j):
                sv = s_scr[...]
                m = jnp.max(sv, axis=1)
                idxc = jnp.where(sv == m[:, None], iota_b, _BIG)
                a = jnp.min(idxc, axis=1)
                cand_v[:, j:j + 1] = m[:, None]
                cand_i[:, j:j + 1] = a[:, None] + gbase
                s_scr[...] = jnp.where(idxc == a[:, None], -jnp.inf, sv)

        cv = jnp.concatenate([out_v_ref[...], cand_v[...]], axis=1)
        ci = jnp.concatenate([out_i_ref[...], cand_i[...]], axis=1)
        iota_c = jax.lax.broadcasted_iota(jnp.int32, cv.shape, 1)
        nv_cols, ni_cols = [], []
        for _ in range(topk):
            m = jnp.max(cv, axis=1)
            idxc = jnp.where(cv == m[:, None], iota_c, _BIG)
            a = jnp.min(idxc, axis=1)
            oh = idxc == a[:, None]
            sel = jnp.sum(jnp.where(oh, ci, 0), axis=1)
            nv_cols.append(m[:, None])
            ni_cols.append(sel[:, None])
            cv = jnp.where(oh, -jnp.inf, cv)
        out_v_ref[...] = jnp.concatenate(nv_cols, axis=1)
        out_i_ref[...] = jnp.concatenate(ni_cols, axis=1)


def _retrieve(qn, keys, blk, interpret=False):
    Q, D = qn.shape
    K, _ = keys.shape
    assert K % blk == 0
    nblk = K // blk
    kfn = functools.partial(_retrieve_kernel, nblk=nblk, blk=blk, topk=TOPK)
    return pl.pallas_call(
        kfn,
        grid=(nblk,),
        in_specs=[
            pl.BlockSpec((Q, D), lambda i: (0, 0)),
            pl.BlockSpec((blk, D), lambda i: (i, 0)),
        ],
        out_specs=[
            pl.BlockSpec((Q, TOPK), lambda i: (0, 0)),
            pl.BlockSpec((Q, TOPK), lambda i: (0, 0)),
        ],
        out_shape=[
            jax.ShapeDtypeStruct((Q, TOPK), jnp.float32),
            jax.ShapeDtypeStruct((Q, TOPK), jnp.int32),
        ],
        scratch_shapes=[
            pltpu.VMEM((Q, blk), jnp.float32),
            pltpu.VMEM((Q, TOPK), jnp.float32),
            pltpu.VMEM((Q, TOPK), jnp.int32),
        ],
        interpret=interpret,
    )(qn, keys)


@jax.jit
def kernel(queries, keys):
    qn = queries / (jnp.linalg.norm(queries, axis=-1, keepdims=True) + 1e-8)
    K = keys.shape[0]
    blk = 2000 if K % 2000 == 0 else K
    return _retrieve(qn, keys, blk)
